# Initial kernel scaffold; baseline (speedup 1.0000x reference)
import jax
import jax.numpy as jnp
from jax.experimental import pallas as pl


def kernel(x_real, x_imag, gamma, beta):
    raise NotImplementedError("write your pallas kernel here")


# speedup vs baseline: 3.4160x; 3.4160x over previous
import functools

import jax
import jax.numpy as jnp
from jax.experimental import pallas as pl
from jax.experimental.pallas import tpu as pltpu

EPS_ = 1e-5


def _stats_body(xr_ref, xi_ref, out_ref):
    xr = xr_ref[0]
    xi = xi_ref[0]
    c = xr.shape[0]

    def colsum(v):
        return jnp.sum(jnp.sum(v, axis=1), axis=-1, keepdims=True)

    sr = colsum(xr)
    si = colsum(xi)
    srr = colsum(xr * xr)
    sii = colsum(xi * xi)
    sri = colsum(xr * xi)
    pad = jnp.zeros((c, 123), jnp.float32)
    out_ref[...] = jnp.concatenate([sr, si, srr, sii, sri, pad], axis=1)[None]


def _coef_body(part_ref, gb_ref, out_ref, *, n):
    psum = jnp.sum(part_ref[...], axis=0)
    c = psum.shape[0]
    sr = psum[:, 0:1]
    si = psum[:, 1:2]
    srr = psum[:, 2:3]
    sii = psum[:, 3:4]
    sri = psum[:, 4:5]
    inv_n = 1.0 / n
    inv_nm1 = 1.0 / (n - 1.0)
    mr = sr * inv_n
    mi = si * inv_n
    va = (srr - sr * mr) * inv_nm1 + EPS_
    vc = (sii - si * mi) * inv_nm1 + EPS_
    vb = (sri - sr * mi) * inv_nm1
    s = jnp.sqrt(va * vc - vb * vb)
    t = jnp.sqrt(va + vc + 2.0 * s)
    inv = 1.0 / (s * t)
    p00 = (vc + s) * inv
    p01 = -vb * inv
    p11 = (va + s) * inv
    g00 = gb_ref[:, 0:1]
    g01 = gb_ref[:, 1:2]
    g10 = gb_ref[:, 2:3]
    g11 = gb_ref[:, 3:4]
    be0 = gb_ref[:, 4:5]
    be1 = gb_ref[:, 5:6]
    a00 = g00 * p00 + g01 * p01
    a01 = g00 * p01 + g01 * p11
    a10 = g10 * p00 + g11 * p01
    a11 = g10 * p01 + g11 * p11
    b0 = be0 - (a00 * mr + a01 * mi)
    b1 = be1 - (a10 * mr + a11 * mi)
    pad = jnp.zeros((c, 122), jnp.float32)
    out_ref[...] = jnp.concatenate([a00, a01, a10, a11, b0, b1, pad], axis=1)


def _apply_body(xr_ref, xi_ref, coef_ref, out_ref):
    xr = xr_ref[0]
    xi = xi_ref[0]
    c, h, w = xr.shape
    a00 = coef_ref[:, 0:1].reshape(c, 1, 1)
    a01 = coef_ref[:, 1:2].reshape(c, 1, 1)
    a10 = coef_ref[:, 2:3].reshape(c, 1, 1)
    a11 = coef_ref[:, 3:4].reshape(c, 1, 1)
    b0 = coef_ref[:, 4:5].reshape(c, 1, 1)
    b1 = coef_ref[:, 5:6].reshape(c, 1, 1)
    yr = a00 * xr + a01 * xi + b0
    yi = a10 * xr + a11 * xi + b1
    lane = jax.lax.broadcasted_iota(jnp.int32, (c, h, w), 2)
    parity = (lane % 2) == 1
    idx_lo = lane // 2
    idx_hi = idx_lo + (w // 2)
    z_lo = jnp.where(parity,
                     jnp.take_along_axis(yi, idx_lo, axis=-1),
                     jnp.take_along_axis(yr, idx_lo, axis=-1))
    z_hi = jnp.where(parity,
                     jnp.take_along_axis(yi, idx_hi, axis=-1),
                     jnp.take_along_axis(yr, idx_hi, axis=-1))
    out_ref[...] = jnp.concatenate([z_lo, z_hi], axis=-1)[None]


def kernel(x_real, x_imag, gamma, beta):
    B, C, H, W = x_real.shape
    n = float(B * H * W)

    partial = pl.pallas_call(
        _stats_body,
        grid=(B,),
        in_specs=[
            pl.BlockSpec((1, C, H, W), lambda b: (b, 0, 0, 0)),
            pl.BlockSpec((1, C, H, W), lambda b: (b, 0, 0, 0)),
        ],
        out_specs=pl.BlockSpec((1, C, 128), lambda b: (b, 0, 0)),
        out_shape=jax.ShapeDtypeStruct((B, C, 128), jnp.float32),
        compiler_params=pltpu.CompilerParams(
            dimension_semantics=("parallel",),
            vmem_limit_bytes=64 * 1024 * 1024,
        ),
    )(x_real, x_imag)

    gb = jnp.concatenate(
        [gamma.reshape(C, 4), beta.reshape(C, 2)], axis=1)
    gb = jnp.pad(gb, ((0, 0), (0, 122)))

    coef = pl.pallas_call(
        functools.partial(_coef_body, n=n),
        in_specs=[
            pl.BlockSpec((B, C, 128), lambda: (0, 0, 0)),
            pl.BlockSpec((C, 128), lambda: (0, 0)),
        ],
        out_specs=pl.BlockSpec((C, 128), lambda: (0, 0)),
        out_shape=jax.ShapeDtypeStruct((C, 128), jnp.float32),
    )(partial, gb)

    y = pl.pallas_call(
        _apply_body,
        grid=(B,),
        in_specs=[
            pl.BlockSpec((1, C, H, W), lambda b: (b, 0, 0, 0)),
            pl.BlockSpec((1, C, H, W), lambda b: (b, 0, 0, 0)),
            pl.BlockSpec((C, 128), lambda b: (0, 0)),
        ],
        out_specs=pl.BlockSpec((1, C, H, 2 * W), lambda b: (b, 0, 0, 0)),
        out_shape=jax.ShapeDtypeStruct((B, C, H, 2 * W), jnp.float32),
        compiler_params=pltpu.CompilerParams(
            dimension_semantics=("parallel",),
            vmem_limit_bytes=96 * 1024 * 1024,
        ),
    )(x_real, x_imag, coef)

    return y.reshape(B, C, H, W, 2)


# speedup vs baseline: 9.2773x; 2.7159x over previous
import functools

import jax
import jax.numpy as jnp
from jax.experimental import pallas as pl
from jax.experimental.pallas import tpu as pltpu

EPS_ = 1e-5


def _stats_body(xr_ref, xi_ref, out_ref):
    xr = xr_ref[0]
    xi = xi_ref[0]
    c = xr.shape[0]

    def colsum(v):
        return jnp.sum(jnp.sum(v, axis=1), axis=-1, keepdims=True)

    sr = colsum(xr)
    si = colsum(xi)
    srr = colsum(xr * xr)
    sii = colsum(xi * xi)
    sri = colsum(xr * xi)
    pad = jnp.zeros((c, 123), jnp.float32)
    out_ref[...] = jnp.concatenate([sr, si, srr, sii, sri, pad], axis=1)[None]


def _coef_body(part_ref, gb_ref, out_ref, *, n):
    psum = jnp.sum(part_ref[...], axis=0)
    c = psum.shape[0]
    sr = psum[:, 0:1]
    si = psum[:, 1:2]
    srr = psum[:, 2:3]
    sii = psum[:, 3:4]
    sri = psum[:, 4:5]
    inv_n = 1.0 / n
    inv_nm1 = 1.0 / (n - 1.0)
    mr = sr * inv_n
    mi = si * inv_n
    va = (srr - sr * mr) * inv_nm1 + EPS_
    vc = (sii - si * mi) * inv_nm1 + EPS_
    vb = (sri - sr * mi) * inv_nm1
    s = jnp.sqrt(va * vc - vb * vb)
    t = jnp.sqrt(va + vc + 2.0 * s)
    inv = 1.0 / (s * t)
    p00 = (vc + s) * inv
    p01 = -vb * inv
    p11 = (va + s) * inv
    g00 = gb_ref[:, 0:1]
    g01 = gb_ref[:, 1:2]
    g10 = gb_ref[:, 2:3]
    g11 = gb_ref[:, 3:4]
    be0 = gb_ref[:, 4:5]
    be1 = gb_ref[:, 5:6]
    a00 = g00 * p00 + g01 * p01
    a01 = g00 * p01 + g01 * p11
    a10 = g10 * p00 + g11 * p01
    a11 = g10 * p01 + g11 * p11
    b0 = be0 - (a00 * mr + a01 * mi)
    b1 = be1 - (a10 * mr + a11 * mi)
    pad = jnp.zeros((c, 122), jnp.float32)
    out_ref[...] = jnp.concatenate([a00, a01, a10, a11, b0, b1, pad], axis=1)


def _apply_body(xr_ref, xi_ref, coef_ref, out_ref):
    xr = xr_ref[0]
    xi = xi_ref[0]
    c, h, w = xr.shape
    a00 = coef_ref[:, 0:1].reshape(c, 1, 1)
    a01 = coef_ref[:, 1:2].reshape(c, 1, 1)
    a10 = coef_ref[:, 2:3].reshape(c, 1, 1)
    a11 = coef_ref[:, 3:4].reshape(c, 1, 1)
    b0 = coef_ref[:, 4:5].reshape(c, 1, 1)
    b1 = coef_ref[:, 5:6].reshape(c, 1, 1)
    yr = a00 * xr + a01 * xi + b0
    yi = a10 * xr + a11 * xi + b1
    g = h // 8
    rv = yr.reshape(c, g, 8, w)
    iv = yi.reshape(c, g, 8, w)
    row = jax.lax.broadcasted_iota(jnp.int32, (c, g, 8, w), 2)
    parity = (row % 2) == 1
    idx_e = row // 2
    idx_o = idx_e + 4
    z_e = jnp.where(parity,
                    jnp.take_along_axis(iv, idx_e, axis=2),
                    jnp.take_along_axis(rv, idx_e, axis=2))
    z_o = jnp.where(parity,
                    jnp.take_along_axis(iv, idx_o, axis=2),
                    jnp.take_along_axis(rv, idx_o, axis=2))
    z = jnp.stack([z_e, z_o], axis=2)
    out_ref[...] = z.reshape(1, c, 2 * h, w)


def kernel(x_real, x_imag, gamma, beta):
    B, C, H, W = x_real.shape
    n = float(B * H * W)

    partial = pl.pallas_call(
        _stats_body,
        grid=(B,),
        in_specs=[
            pl.BlockSpec((1, C, H, W), lambda b: (b, 0, 0, 0)),
            pl.BlockSpec((1, C, H, W), lambda b: (b, 0, 0, 0)),
        ],
        out_specs=pl.BlockSpec((1, C, 128), lambda b: (b, 0, 0)),
        out_shape=jax.ShapeDtypeStruct((B, C, 128), jnp.float32),
        compiler_params=pltpu.CompilerParams(
            dimension_semantics=("parallel",),
            vmem_limit_bytes=64 * 1024 * 1024,
        ),
    )(x_real, x_imag)

    gb = jnp.concatenate(
        [gamma.reshape(C, 4), beta.reshape(C, 2)], axis=1)
    gb = jnp.pad(gb, ((0, 0), (0, 122)))

    coef = pl.pallas_call(
        functools.partial(_coef_body, n=n),
        in_specs=[
            pl.BlockSpec((B, C, 128), lambda: (0, 0, 0)),
            pl.BlockSpec((C, 128), lambda: (0, 0)),
        ],
        out_specs=pl.BlockSpec((C, 128), lambda: (0, 0)),
        out_shape=jax.ShapeDtypeStruct((C, 128), jnp.float32),
    )(partial, gb)

    y = pl.pallas_call(
        _apply_body,
        grid=(B,),
        in_specs=[
            pl.BlockSpec((1, C, H, W), lambda b: (b, 0, 0, 0)),
            pl.BlockSpec((1, C, H, W), lambda b: (b, 0, 0, 0)),
            pl.BlockSpec((C, 128), lambda b: (0, 0)),
        ],
        out_specs=pl.BlockSpec((1, C, 2 * H, W), lambda b: (b, 0, 0, 0)),
        out_shape=jax.ShapeDtypeStruct((B, C, 2 * H, W), jnp.float32),
        compiler_params=pltpu.CompilerParams(
            dimension_semantics=("parallel",),
            vmem_limit_bytes=96 * 1024 * 1024,
        ),
    )(x_real, x_imag, coef)

    return jnp.swapaxes(y.reshape(B, C, H, 2, W), 3, 4)


# speedup vs baseline: 9.4682x; 1.0206x over previous
import functools

import jax
import jax.numpy as jnp
from jax.experimental import pallas as pl
from jax.experimental.pallas import tpu as pltpu

EPS_ = 1e-5


def _stats_body(xr_ref, xi_ref, out_ref):
    xr = xr_ref[...]
    xi = xi_ref[...]
    c = xr.shape[1]

    def colsum(v):
        return jnp.sum(jnp.sum(jnp.sum(v, axis=2), axis=0), axis=-1,
                       keepdims=True)

    sr = colsum(xr)
    si = colsum(xi)
    srr = colsum(xr * xr)
    sii = colsum(xi * xi)
    sri = colsum(xr * xi)
    pad = jnp.zeros((c, 123), jnp.float32)
    out_ref[...] = jnp.concatenate([sr, si, srr, sii, sri, pad], axis=1)[None]


def _coefs(psum, gb, n):
    sr = psum[:, 0:1]
    si = psum[:, 1:2]
    srr = psum[:, 2:3]
    sii = psum[:, 3:4]
    sri = psum[:, 4:5]
    inv_n = 1.0 / n
    inv_nm1 = 1.0 / (n - 1.0)
    mr = sr * inv_n
    mi = si * inv_n
    va = (srr - sr * mr) * inv_nm1 + EPS_
    vc = (sii - si * mi) * inv_nm1 + EPS_
    vb = (sri - sr * mi) * inv_nm1
    s = jnp.sqrt(va * vc - vb * vb)
    t = jnp.sqrt(va + vc + 2.0 * s)
    inv = 1.0 / (s * t)
    p00 = (vc + s) * inv
    p01 = -vb * inv
    p11 = (va + s) * inv
    g00 = gb[:, 0:1]
    g01 = gb[:, 1:2]
    g10 = gb[:, 2:3]
    g11 = gb[:, 3:4]
    be0 = gb[:, 4:5]
    be1 = gb[:, 5:6]
    a00 = g00 * p00 + g01 * p01
    a01 = g00 * p01 + g01 * p11
    a10 = g10 * p00 + g11 * p01
    a11 = g10 * p01 + g11 * p11
    b0 = be0 - (a00 * mr + a01 * mi)
    b1 = be1 - (a10 * mr + a11 * mi)
    return a00, a01, a10, a11, b0, b1


def _apply_body(xr_ref, xi_ref, part_ref, gb_ref, out_ref, *, n):
    xr = xr_ref[0]
    xi = xi_ref[0]
    c, h, w = xr.shape
    psum = jnp.sum(part_ref[...], axis=0)
    a00, a01, a10, a11, b0, b1 = (
        v.reshape(c, 1, 1) for v in _coefs(psum, gb_ref[...], n))
    yr = a00 * xr + a01 * xi + b0
    yi = a10 * xr + a11 * xi + b1
    g = h // 8
    rv = yr.reshape(c, g, 8, w)
    iv = yi.reshape(c, g, 8, w)
    row = jax.lax.broadcasted_iota(jnp.int32, (c, g, 8, w), 2)
    parity = (row % 2) == 1
    idx_e = row // 2
    idx_o = idx_e + 4
    z_e = jnp.where(parity,
                    jnp.take_along_axis(iv, idx_e, axis=2),
                    jnp.take_along_axis(rv, idx_e, axis=2))
    z_o = jnp.where(parity,
                    jnp.take_along_axis(iv, idx_o, axis=2),
                    jnp.take_along_axis(rv, idx_o, axis=2))
    z = jnp.stack([z_e, z_o], axis=2)
    out_ref[...] = z.reshape(1, c, 2 * h, w)


def kernel(x_real, x_imag, gamma, beta):
    B, C, H, W = x_real.shape
    n = float(B * H * W)

    BB = 2
    nb = B // BB
    partial = pl.pallas_call(
        _stats_body,
        grid=(nb,),
        in_specs=[
            pl.BlockSpec((BB, C, H, W), lambda b: (b, 0, 0, 0)),
            pl.BlockSpec((BB, C, H, W), lambda b: (b, 0, 0, 0)),
        ],
        out_specs=pl.BlockSpec((1, C, 128), lambda b: (b, 0, 0)),
        out_shape=jax.ShapeDtypeStruct((nb, C, 128), jnp.float32),
        compiler_params=pltpu.CompilerParams(
            dimension_semantics=("parallel",),
            vmem_limit_bytes=64 * 1024 * 1024,
        ),
    )(x_real, x_imag)

    gb = jnp.concatenate(
        [gamma.reshape(C, 4), beta.reshape(C, 2)], axis=1)
    gb = jnp.pad(gb, ((0, 0), (0, 122)))

    y = pl.pallas_call(
        functools.partial(_apply_body, n=n),
        grid=(B,),
        in_specs=[
            pl.BlockSpec((1, C, H, W), lambda b: (b, 0, 0, 0)),
            pl.BlockSpec((1, C, H, W), lambda b: (b, 0, 0, 0)),
            pl.BlockSpec((nb, C, 128), lambda b: (0, 0, 0)),
            pl.BlockSpec((C, 128), lambda b: (0, 0)),
        ],
        out_specs=pl.BlockSpec((1, C, 2 * H, W), lambda b: (b, 0, 0, 0)),
        out_shape=jax.ShapeDtypeStruct((B, C, 2 * H, W), jnp.float32),
        compiler_params=pltpu.CompilerParams(
            dimension_semantics=("parallel",),
            vmem_limit_bytes=96 * 1024 * 1024,
        ),
    )(x_real, x_imag, partial, gb)

    return jnp.swapaxes(y.reshape(B, C, H, 2, W), 3, 4)
